# baseline (speedup 1.0000x reference)
import functools

import jax
import jax.numpy as jnp
from jax import lax
from jax.experimental import pallas as pl
from jax.experimental.pallas import tpu as pltpu
from jax.experimental.pallas import tpu_sc as plsc

_E = 64
_T = 32768
_ALPHA = 0.02
_NC, _NS, _L = 2, 16, 16
_NW = _NC * _NS
_RPW = _T // _NW
_EV = _E // _L
_CHUNK = 128
_NCH = _RPW // _CHUNK
_NBUF = 4
_RU = 8

_mesh = plsc.VectorSubcoreMesh(core_axis_name="c", subcore_axis_name="s",
                               num_cores=_NC, num_subcores=_NS)


@functools.partial(
    pl.kernel,
    out_type=(
        jax.ShapeDtypeStruct((_NW, _E), jnp.float32),
        jax.ShapeDtypeStruct((_NW, _E), jnp.float32),
    ),
    mesh=_mesh,
    scratch_types=[
        pltpu.VMEM((_NBUF, _CHUNK, _E), jnp.float32),
        pltpu.VMEM((_RPW,), jnp.int32),
        pltpu.VMEM((_L * _E,), jnp.float32),
        pltpu.VMEM((_E,), jnp.float32),
        pltpu.VMEM((_E,), jnp.float32),
        pltpu.SemaphoreType.DMA,
        pltpu.SemaphoreType.DMA,
        pltpu.SemaphoreType.DMA,
        pltpu.SemaphoreType.DMA,
    ],
    compiler_params=pltpu.CompilerParams(needs_layout_passes=False),
)
def _partials(probs_hbm, idx_hbm, colsum_out, counts_out,
              probs_v, idx_v, hist_v, cs_v, cnt_v, sem0, sem1, sem2, sem3):
    sems = (sem0, sem1, sem2, sem3)
    wid = lax.axis_index("s") * _NC + lax.axis_index("c")
    base = wid * _RPW

    cps = [
        pltpu.async_copy(probs_hbm.at[pl.ds(base + b * _CHUNK, _CHUNK)],
                         probs_v.at[b], sems[b])
        for b in range(_NBUF)
    ]
    pltpu.sync_copy(idx_hbm.at[pl.ds(base, _RPW)], idx_v)

    zero16 = jnp.zeros((_L,), jnp.float32)

    def zbody(i, c):
        hist_v[pl.ds(i * _L, _L)] = zero16
        return c
    lax.fori_loop(0, _E, zbody, 0)

    lane = lax.iota(jnp.int32, _L) * _E
    ones = jnp.ones((_L,), jnp.float32)

    def hbody(i, c):
        idx = idx_v[pl.ds(i * _L, _L)]
        plsc.addupdate_scatter(hist_v, [lane + idx], ones)
        return c
    lax.fori_loop(0, _RPW // _L, hbody, 0)

    def cbody(l, acc):
        return tuple(acc[j] + hist_v[pl.ds(l * _E + j * _L, _L)]
                     for j in range(_EV))
    cnt = lax.fori_loop(0, _L, cbody, (zero16,) * _EV)
    for j in range(_EV):
        cnt_v[pl.ds(j * _L, _L)] = cnt[j]

    acc = (zero16,) * (2 * _EV)
    for k in range(_NCH):
        b = k % _NBUF
        cps[b].wait()

        def rbody(r, a, _b=b):
            a = list(a)
            for rr in range(_RU):
                for j in range(_EV):
                    s = (rr % 2) * _EV + j
                    a[s] = a[s] + probs_v[_b, r + rr, pl.ds(j * _L, _L)]
            return tuple(a)
        acc = plsc.parallel_loop(0, _CHUNK, step=_RU, carry=acc)(rbody)
        if k + _NBUF < _NCH:
            cps[b] = pltpu.async_copy(
                probs_hbm.at[pl.ds(base + (k + _NBUF) * _CHUNK, _CHUNK)],
                probs_v.at[b], sems[b])
    for j in range(_EV):
        cs_v[pl.ds(j * _L, _L)] = acc[j] + acc[_EV + j]

    pltpu.sync_copy(cs_v, colsum_out.at[wid])
    pltpu.sync_copy(cnt_v, counts_out.at[wid])


def kernel(router_probs, expert_idx):
    pc, ph = _partials(router_probs, expert_idx)
    colsum = pc.sum(axis=0)
    counts = ph.sum(axis=0)
    total = counts.sum()
    f_i = counts / jnp.where(total < 1e-9, 1.0, total)
    p_i = colsum / router_probs.shape[0]
    loss = _ALPHA * _E * (f_i * p_i).sum()
    return jnp.where(total < 1e-9, 0.0, loss)


# speedup vs baseline: 1.1239x; 1.1239x over previous
import functools

import jax
import jax.numpy as jnp
from jax import lax
from jax.experimental import pallas as pl
from jax.experimental.pallas import tpu as pltpu
from jax.experimental.pallas import tpu_sc as plsc

_E = 64
_T = 32768
_ALPHA = 0.02
_NC, _NS, _L = 2, 16, 16
_NW = _NC * _NS
_IPW = _T // _NW
_EV = _E // _L

_BLK = 4096
_GRID = _T // _BLK

_mesh = plsc.VectorSubcoreMesh(core_axis_name="c", subcore_axis_name="s",
                               num_cores=_NC, num_subcores=_NS)


@functools.partial(
    pl.kernel,
    out_type=jax.ShapeDtypeStruct((_NW, _E), jnp.float32),
    mesh=_mesh,
    scratch_types=[
        pltpu.VMEM((_IPW,), jnp.int32),
        pltpu.VMEM((_L * _E,), jnp.float32),
        pltpu.VMEM((_E,), jnp.float32),
    ],
    compiler_params=pltpu.CompilerParams(needs_layout_passes=False),
)
def _hist(idx_hbm, counts_out, idx_v, hist_v, cnt_v):
    wid = lax.axis_index("s") * _NC + lax.axis_index("c")
    base = wid * _IPW
    pltpu.sync_copy(idx_hbm.at[pl.ds(base, _IPW)], idx_v)

    zero16 = jnp.zeros((_L,), jnp.float32)

    def zbody(i, c):
        hist_v[pl.ds(i * _L, _L)] = zero16
        return c
    lax.fori_loop(0, _E, zbody, 0)

    lane = lax.iota(jnp.int32, _L) * _E
    ones = jnp.ones((_L,), jnp.float32)

    def hbody(i, c):
        idx = idx_v[pl.ds(i * _L, _L)]
        plsc.addupdate_scatter(hist_v, [lane + idx], ones)
        return c
    lax.fori_loop(0, _IPW // _L, hbody, 0)

    def cbody(l, acc):
        return tuple(acc[j] + hist_v[pl.ds(l * _E + j * _L, _L)]
                     for j in range(_EV))
    cnt = lax.fori_loop(0, _L, cbody, (zero16,) * _EV)
    for j in range(_EV):
        cnt_v[pl.ds(j * _L, _L)] = cnt[j]

    pltpu.sync_copy(cnt_v, counts_out.at[wid])


def _loss_body(probs_ref, cnt_ref, out_ref, acc_ref):
    i = pl.program_id(0)

    @pl.when(i == 0)
    def _init():
        acc_ref[...] = jnp.zeros_like(acc_ref)

    acc_ref[...] += jnp.sum(probs_ref[...], axis=0, keepdims=True)

    @pl.when(i == _GRID - 1)
    def _fin():
        colsum = acc_ref[0, :]
        counts = jnp.sum(cnt_ref[...], axis=0)
        total = jnp.sum(counts)
        f_i = counts / jnp.where(total < 1e-9, 1.0, total)
        p_i = colsum / jnp.float32(_T)
        loss = _ALPHA * _E * jnp.sum(f_i * p_i)
        out_ref[0, 0] = jnp.where(total < 1e-9, 0.0, loss)


_loss = pl.pallas_call(
    _loss_body,
    grid=(_GRID,),
    in_specs=[
        pl.BlockSpec((_BLK, _E), lambda i: (i, 0)),
        pl.BlockSpec((_NW, _E), lambda i: (0, 0)),
    ],
    out_specs=pl.BlockSpec(memory_space=pltpu.SMEM),
    out_shape=jax.ShapeDtypeStruct((1, 1), jnp.float32),
    scratch_shapes=[pltpu.VMEM((1, _E), jnp.float32)],
    compiler_params=pltpu.CompilerParams(
        dimension_semantics=("arbitrary",)),
)


def kernel(router_probs, expert_idx):
    counts_part = _hist(expert_idx)
    return _loss(router_probs, counts_part)[0, 0]


# speedup vs baseline: 1.6005x; 1.4241x over previous
import functools

import jax
import jax.numpy as jnp
from jax import lax
from jax.experimental import pallas as pl
from jax.experimental.pallas import tpu as pltpu
from jax.experimental.pallas import tpu_sc as plsc

_E = 64
_T = 32768
_ALPHA = 0.02
_NC, _NS, _L = 2, 16, 16
_NW = _NC * _NS
_IPW = _T // _NW
_EV = _E // _L

_BLKT = 4096
_GRID = _T // _BLKT
_LANES = 128

_mesh = plsc.VectorSubcoreMesh(core_axis_name="c", subcore_axis_name="s",
                               num_cores=_NC, num_subcores=_NS)


@functools.partial(
    pl.kernel,
    out_type=jax.ShapeDtypeStruct((_NW, _E), jnp.float32),
    mesh=_mesh,
    scratch_types=[
        pltpu.VMEM((_IPW,), jnp.int32),
        pltpu.VMEM((_L * _E,), jnp.float32),
        pltpu.VMEM((_E,), jnp.float32),
    ],
    compiler_params=pltpu.CompilerParams(needs_layout_passes=False),
)
def _hist(idx_hbm, counts_out, idx_v, hist_v, cnt_v):
    wid = lax.axis_index("s") * _NC + lax.axis_index("c")
    base = wid * _IPW
    pltpu.sync_copy(idx_hbm.at[pl.ds(base, _IPW)], idx_v)

    zero16 = jnp.zeros((_L,), jnp.float32)

    def zbody(i, c):
        hist_v[pl.ds(i * _L, _L)] = zero16
        return c
    lax.fori_loop(0, _E, zbody, 0)

    lane = lax.iota(jnp.int32, _L) * _E
    ones = jnp.ones((_L,), jnp.float32)

    def hbody(i, c):
        idx = idx_v[pl.ds(i * _L, _L)]
        plsc.addupdate_scatter(hist_v, [lane + idx], ones)
        return c
    lax.fori_loop(0, _IPW // _L, hbody, 0)

    def cbody(l, acc):
        return tuple(acc[j] + hist_v[pl.ds(l * _E + j * _L, _L)]
                     for j in range(_EV))
    cnt = lax.fori_loop(0, _L, cbody, (zero16,) * _EV)
    for j in range(_EV):
        cnt_v[pl.ds(j * _L, _L)] = cnt[j]

    pltpu.sync_copy(cnt_v, counts_out.at[wid])


def _colsum_body(probs_ref, out_ref, acc_ref):
    i = pl.program_id(0)

    @pl.when(i == 0)
    def _init():
        acc_ref[...] = jnp.zeros_like(acc_ref)

    a = acc_ref[...]
    for j in range(_BLKT // _LANES):
        a = a + probs_ref[:, pl.ds(j * _LANES, _LANES)]
    acc_ref[...] = a

    @pl.when(i == _GRID - 1)
    def _fin():
        out_ref[...] = jnp.sum(acc_ref[...], axis=1, keepdims=True).T


_colsum = pl.pallas_call(
    _colsum_body,
    grid=(_GRID,),
    in_specs=[pl.BlockSpec((_E, _BLKT), lambda i: (0, i))],
    out_specs=pl.BlockSpec((1, _E), lambda i: (0, 0)),
    out_shape=jax.ShapeDtypeStruct((1, _E), jnp.float32),
    scratch_shapes=[pltpu.VMEM((_E, _LANES), jnp.float32)],
    compiler_params=pltpu.CompilerParams(
        dimension_semantics=("arbitrary",)),
)


def kernel(router_probs, expert_idx):
    counts_part = _hist(expert_idx)
    colsum = _colsum(router_probs.T)[0]
    counts = jnp.sum(counts_part, axis=0)
    total = jnp.sum(counts)
    f_i = counts / jnp.where(total < 1e-9, 1.0, total)
    p_i = colsum / jnp.float32(_T)
    loss = _ALPHA * _E * jnp.sum(f_i * p_i)
    return jnp.where(total < 1e-9, 0.0, loss)


# speedup vs baseline: 1.7322x; 1.0823x over previous
import functools

import jax
import jax.numpy as jnp
from jax import lax
from jax.experimental import pallas as pl
from jax.experimental.pallas import tpu as pltpu
from jax.experimental.pallas import tpu_sc as plsc

_E = 64
_T = 32768
_ALPHA = 0.02
_NC, _NS, _L = 2, 16, 16
_NW = _NC * _NS
_IPW = _T // _NW
_EV = _E // _L

_BLKT = 8192
_GRID = _T // _BLKT
_LANES = 128

_mesh = plsc.VectorSubcoreMesh(core_axis_name="c", subcore_axis_name="s",
                               num_cores=_NC, num_subcores=_NS)


@functools.partial(
    pl.kernel,
    out_type=jax.ShapeDtypeStruct((_NW, _E), jnp.float32),
    mesh=_mesh,
    scratch_types=[
        pltpu.VMEM((_IPW,), jnp.int32),
        pltpu.VMEM((_L * _E,), jnp.float32),
        pltpu.VMEM((_E,), jnp.float32),
    ],
    compiler_params=pltpu.CompilerParams(needs_layout_passes=False),
)
def _hist(idx_hbm, counts_out, idx_v, hist_v, cnt_v):
    wid = lax.axis_index("s") * _NC + lax.axis_index("c")
    base = wid * _IPW
    pltpu.sync_copy(idx_hbm.at[pl.ds(base, _IPW)], idx_v)

    zero16 = jnp.zeros((_L,), jnp.float32)

    def zbody(i, c):
        hist_v[pl.ds(i * _L, _L)] = zero16
        return c
    lax.fori_loop(0, _E, zbody, 0)

    lane = lax.iota(jnp.int32, _L) * _E
    ones = jnp.ones((_L,), jnp.float32)

    def hbody(i, c):
        idx = idx_v[pl.ds(i * _L, _L)]
        plsc.addupdate_scatter(hist_v, [lane + idx], ones)
        return c
    lax.fori_loop(0, _IPW // _L, hbody, 0)

    def cbody(l, acc):
        return tuple(acc[j] + hist_v[pl.ds(l * _E + j * _L, _L)]
                     for j in range(_EV))
    cnt = lax.fori_loop(0, _L, cbody, (zero16,) * _EV)
    for j in range(_EV):
        cnt_v[pl.ds(j * _L, _L)] = cnt[j]

    pltpu.sync_copy(cnt_v, counts_out.at[wid])


def _colsum_body(probs_ref, out_ref, acc_ref):
    i = pl.program_id(0)

    @pl.when(i == 0)
    def _init():
        acc_ref[...] = jnp.zeros_like(acc_ref)

    a = acc_ref[...]
    for j in range(_BLKT // _LANES):
        a = a + probs_ref[:, pl.ds(j * _LANES, _LANES)]
    acc_ref[...] = a

    @pl.when(i == _GRID - 1)
    def _fin():
        out_ref[...] = jnp.sum(acc_ref[...], axis=1, keepdims=True).T


_colsum = pl.pallas_call(
    _colsum_body,
    grid=(_GRID,),
    in_specs=[pl.BlockSpec((_E, _BLKT), lambda i: (0, i))],
    out_specs=pl.BlockSpec((1, _E), lambda i: (0, 0)),
    out_shape=jax.ShapeDtypeStruct((1, _E), jnp.float32),
    scratch_shapes=[pltpu.VMEM((_E, _LANES), jnp.float32)],
    compiler_params=pltpu.CompilerParams(
        dimension_semantics=("arbitrary",)),
)


def kernel(router_probs, expert_idx):
    counts_part = _hist(expert_idx)
    colsum = _colsum(router_probs.T)
    weighted = counts_part * colsum
    s = jnp.sum(weighted)
    total = jnp.sum(counts_part)
    loss = (_ALPHA * _E / _T) * s / jnp.where(total < 1e-9, 1.0, total)
    return jnp.where(total < 1e-9, 0.0, loss)


# speedup vs baseline: 1.7959x; 1.0367x over previous
import functools

import jax
import jax.numpy as jnp
from jax import lax
from jax.experimental import pallas as pl
from jax.experimental.pallas import tpu as pltpu
from jax.experimental.pallas import tpu_sc as plsc

_E = 64
_T = 32768
_ALPHA = 0.02
_NC, _NS, _L = 1, 16, 16
_NW = _NC * _NS
_IPW = _T // _NW
_EV = _E // _L

_BLKT = 8192
_GRID = _T // _BLKT
_LANES = 128

_mesh = plsc.VectorSubcoreMesh(core_axis_name="c", subcore_axis_name="s",
                               num_cores=_NC, num_subcores=_NS)


@functools.partial(
    pl.kernel,
    out_type=jax.ShapeDtypeStruct((_NW, _E), jnp.float32),
    mesh=_mesh,
    scratch_types=[
        pltpu.VMEM((_IPW,), jnp.int32),
        pltpu.VMEM((_L * _E,), jnp.float32),
        pltpu.VMEM((_E,), jnp.float32),
    ],
    compiler_params=pltpu.CompilerParams(needs_layout_passes=False),
)
def _hist(idx_hbm, counts_out, idx_v, hist_v, cnt_v):
    wid = lax.axis_index("s") * _NC + lax.axis_index("c")
    base = wid * _IPW
    pltpu.sync_copy(idx_hbm.at[pl.ds(base, _IPW)], idx_v)

    zero16 = jnp.zeros((_L,), jnp.float32)

    def zbody(i, c):
        hist_v[pl.ds(i * _L, _L)] = zero16
        return c
    lax.fori_loop(0, _E, zbody, 0)

    lane = lax.iota(jnp.int32, _L) * _E
    ones = jnp.ones((_L,), jnp.float32)

    def hbody(i, c):
        idx = idx_v[pl.ds(i * _L, _L)]
        plsc.addupdate_scatter(hist_v, [lane + idx], ones)
        return c
    lax.fori_loop(0, _IPW // _L, hbody, 0)

    def cbody(l, acc):
        return tuple(acc[j] + hist_v[pl.ds(l * _E + j * _L, _L)]
                     for j in range(_EV))
    cnt = lax.fori_loop(0, _L, cbody, (zero16,) * _EV)
    for j in range(_EV):
        cnt_v[pl.ds(j * _L, _L)] = cnt[j]

    pltpu.sync_copy(cnt_v, counts_out.at[wid])


def _colsum_body(probs_ref, out_ref, acc_ref):
    i = pl.program_id(0)

    @pl.when(i == 0)
    def _init():
        acc_ref[...] = jnp.zeros_like(acc_ref)

    a = acc_ref[...]
    for j in range(_BLKT // _LANES):
        a = a + probs_ref[:, pl.ds(j * _LANES, _LANES)]
    acc_ref[...] = a

    @pl.when(i == _GRID - 1)
    def _fin():
        out_ref[...] = jnp.sum(acc_ref[...], axis=1, keepdims=True).T


_colsum = pl.pallas_call(
    _colsum_body,
    grid=(_GRID,),
    in_specs=[pl.BlockSpec((_E, _BLKT), lambda i: (0, i))],
    out_specs=pl.BlockSpec((1, _E), lambda i: (0, 0)),
    out_shape=jax.ShapeDtypeStruct((1, _E), jnp.float32),
    scratch_shapes=[pltpu.VMEM((_E, _LANES), jnp.float32)],
    compiler_params=pltpu.CompilerParams(
        dimension_semantics=("arbitrary",)),
)


def kernel(router_probs, expert_idx):
    counts_part = _hist(expert_idx)
    colsum = _colsum(router_probs.T)
    weighted = counts_part * colsum
    s = jnp.sum(weighted)
    total = jnp.sum(counts_part)
    loss = (_ALPHA * _E / _T) * s / jnp.where(total < 1e-9, 1.0, total)
    return jnp.where(total < 1e-9, 0.0, loss)


# speedup vs baseline: 1.7981x; 1.0012x over previous
import functools

import jax
import jax.numpy as jnp
from jax import lax
from jax.experimental import pallas as pl
from jax.experimental.pallas import tpu as pltpu
from jax.experimental.pallas import tpu_sc as plsc

_E = 64
_T = 32768
_ALPHA = 0.02
_NC, _NS, _L = 1, 16, 16
_NW = _NC * _NS
_IPW = _T // _NW
_EV = _E // _L

_BLKT = 8192
_GRID = _T // _BLKT
_LANES = 128

_mesh = plsc.VectorSubcoreMesh(core_axis_name="c", subcore_axis_name="s",
                               num_cores=_NC, num_subcores=_NS)


@functools.partial(
    pl.kernel,
    out_type=jax.ShapeDtypeStruct((_NW, _E), jnp.float32),
    mesh=_mesh,
    scratch_types=[
        pltpu.VMEM((_IPW,), jnp.int32),
        pltpu.VMEM((_L * _E,), jnp.float32),
        pltpu.VMEM((_E,), jnp.float32),
    ],
    compiler_params=pltpu.CompilerParams(needs_layout_passes=False),
)
def _hist(idx_hbm, counts_out, idx_v, hist_v, cnt_v):
    wid = lax.axis_index("s") * _NC + lax.axis_index("c")
    base = wid * _IPW
    pltpu.sync_copy(idx_hbm.at[pl.ds(base, _IPW)], idx_v)

    zero16 = jnp.zeros((_L,), jnp.float32)

    def zbody(i, c):
        hist_v[pl.ds(i * _L, _L)] = zero16
        return c
    lax.fori_loop(0, _E, zbody, 0)

    lane = lax.iota(jnp.int32, _L) * _E
    ones = jnp.ones((_L,), jnp.float32)

    def hbody(i, c):
        idx = idx_v[pl.ds(i * _L, _L)]
        plsc.addupdate_scatter(hist_v, [lane + idx], ones)
        return c
    lax.fori_loop(0, _IPW // _L, hbody, 0)

    def cbody(l, acc):
        return tuple(acc[j] + hist_v[pl.ds(l * _E + j * _L, _L)]
                     for j in range(_EV))
    cnt = lax.fori_loop(0, _L, cbody, (zero16,) * _EV)
    for j in range(_EV):
        cnt_v[pl.ds(j * _L, _L)] = cnt[j]

    pltpu.sync_copy(cnt_v, counts_out.at[wid])


def _colsum_body(probs_ref, out_ref):
    acc = jnp.zeros((_E, _LANES), jnp.float32)
    for j in range(_T // _LANES):
        acc = acc + probs_ref[:, j * _LANES:(j + 1) * _LANES]
    out_ref[...] = jnp.sum(acc, axis=1, keepdims=True).T


_colsum = pl.pallas_call(
    _colsum_body,
    in_specs=[pl.BlockSpec(memory_space=pltpu.VMEM)],
    out_specs=pl.BlockSpec(memory_space=pltpu.VMEM),
    out_shape=jax.ShapeDtypeStruct((1, _E), jnp.float32),
)


def kernel(router_probs, expert_idx):
    counts_part = _hist(expert_idx)
    colsum = _colsum(router_probs.T)
    weighted = counts_part * colsum
    s = jnp.sum(weighted)
    total = jnp.sum(counts_part)
    loss = (_ALPHA * _E / _T) * s / jnp.where(total < 1e-9, 1.0, total)
    return jnp.where(total < 1e-9, 0.0, loss)
